# baseline (speedup 1.0000x reference)
import functools

import jax
import jax.numpy as jnp
from jax import lax
from jax.experimental import pallas as pl
from jax.experimental.pallas import tpu as pltpu
from jax.experimental.pallas import tpu_sc as plsc

VOCAB = 1000
EMB_D = 128
BATCH = 16384
CHUNK = 128


@functools.cache
def _build():
    info = plsc.get_sparse_core_info()
    nw = info.num_subcores
    b_per_w = BATCH // nw
    half = b_per_w // 2
    n_chunks = half // CHUNK
    mesh = plsc.VectorSubcoreMesh(
        core_axis_name="c", subcore_axis_name="s", num_cores=1
    )

    @functools.partial(
        pl.kernel,
        mesh=mesh,
        out_type=jax.ShapeDtypeStruct((BATCH, EMB_D), jnp.float32),
        scratch_types=[
            pltpu.VMEM((2 * n_chunks, CHUNK), jnp.int32),
            pltpu.VMEM((half, EMB_D), jnp.float32),
            pltpu.SemaphoreType.DMA,
        ],
    )
    def emb_kernel(idx_hbm, table_hbm, out_hbm, idx_v, rows_v, sem):
        wid = lax.axis_index("s")
        pltpu.sync_copy(idx_hbm.at[wid], idx_v)
        for h in range(2):
            base = wid * b_per_w + h * half
            gathers = []
            for j in range(n_chunks):
                gathers.append(
                    pltpu.async_copy(
                        table_hbm.at[idx_v.at[h * n_chunks + j]],
                        rows_v.at[pl.ds(j * CHUNK, CHUNK)],
                        sem,
                    )
                )
            for c in gathers:
                c.wait()
            pltpu.sync_copy(rows_v, out_hbm.at[pl.ds(base, half)])

    return emb_kernel, nw


def kernel(x, table):
    emb_kernel, nw = _build()
    idx = x.astype(jnp.int32).reshape(nw, (BATCH // nw) // CHUNK, CHUNK)
    return emb_kernel(idx, table)


# speedup vs baseline: 1.1481x; 1.1481x over previous
import functools

import jax
import jax.numpy as jnp
from jax import lax
from jax.experimental import pallas as pl
from jax.experimental.pallas import tpu as pltpu
from jax.experimental.pallas import tpu_sc as plsc

VOCAB = 1000
EMB_D = 128
BATCH = 16384
CHUNK = 128


@functools.cache
def _build():
    info = plsc.get_sparse_core_info()
    nc = info.num_cores
    nw = nc * info.num_subcores
    b_per_w = BATCH // nw
    n_chunks = b_per_w // CHUNK
    mesh = plsc.VectorSubcoreMesh(core_axis_name="c", subcore_axis_name="s")

    @functools.partial(
        pl.kernel,
        mesh=mesh,
        out_type=jax.ShapeDtypeStruct((BATCH, EMB_D), jnp.float32),
        scratch_types=[
            pltpu.VMEM((n_chunks, CHUNK), jnp.int32),
            pltpu.VMEM((b_per_w, EMB_D), jnp.float32),
            pltpu.VMEM_SHARED((VOCAB, EMB_D), jnp.float32),
            pltpu.SemaphoreType.DMA,
        ],
    )
    def emb_kernel(idx_hbm, table_hbm, out_hbm, idx_v, rows_v, table_sp, sem):
        sid = lax.axis_index("s")
        wid = sid * nc + lax.axis_index("c")
        base = wid * b_per_w

        @pl.when(sid == 0)
        def _stage():
            pltpu.sync_copy(table_hbm, table_sp)

        pltpu.sync_copy(idx_hbm.at[wid], idx_v)
        plsc.subcore_barrier()
        gathers = []
        for j in range(n_chunks):
            gathers.append(
                pltpu.async_copy(
                    table_sp.at[idx_v.at[j]],
                    rows_v.at[pl.ds(j * CHUNK, CHUNK)],
                    sem,
                )
            )
        for c in gathers:
            c.wait()
        pltpu.sync_copy(rows_v, out_hbm.at[pl.ds(base, b_per_w)])

    return emb_kernel, nw, n_chunks


def kernel(x, table):
    emb_kernel, nw, n_chunks = _build()
    idx = x.astype(jnp.int32).reshape(nw, n_chunks, CHUNK)
    return emb_kernel(idx, table)


# speedup vs baseline: 1.2013x; 1.0464x over previous
import functools

import jax
import jax.numpy as jnp
from jax import lax
from jax.experimental import pallas as pl
from jax.experimental.pallas import tpu as pltpu
from jax.experimental.pallas import tpu_sc as plsc

VOCAB = 1000
EMB_D = 128
BATCH = 16384
CHUNK = 128


@functools.cache
def _build():
    info = plsc.get_sparse_core_info()
    nc = info.num_cores
    nw = nc * info.num_subcores
    b_per_w = BATCH // nw
    n_chunks = b_per_w // CHUNK
    mesh = plsc.VectorSubcoreMesh(core_axis_name="c", subcore_axis_name="s")

    @functools.partial(
        pl.kernel,
        mesh=mesh,
        out_type=jax.ShapeDtypeStruct((BATCH, EMB_D), jnp.float32),
        scratch_types=[
            pltpu.VMEM((n_chunks, CHUNK), jnp.int32),
            pltpu.VMEM((b_per_w, EMB_D), jnp.float32),
            pltpu.VMEM_SHARED((VOCAB, EMB_D), jnp.float32),
            pltpu.SemaphoreType.DMA,
            pltpu.SemaphoreType.DMA,
        ],
    )
    def emb_kernel(idx_hbm, table_hbm, out_hbm, idx_v, rows_v, table_sp, sem, ssem):
        sid = lax.axis_index("s")
        wid = sid * nc + lax.axis_index("c")
        base = wid * b_per_w

        @pl.when(sid == 0)
        def _stage():
            pltpu.sync_copy(table_hbm, table_sp)

        pltpu.sync_copy(idx_hbm.at[wid], idx_v)
        plsc.subcore_barrier()
        gathers = []
        for j in range(n_chunks):
            gathers.append(
                pltpu.async_copy(
                    table_sp.at[idx_v.at[j]],
                    rows_v.at[pl.ds(j * CHUNK, CHUNK)],
                    sem,
                )
            )
        stores = []
        for j in range(n_chunks):
            gathers[j].wait()
            stores.append(
                pltpu.async_copy(
                    rows_v.at[pl.ds(j * CHUNK, CHUNK)],
                    out_hbm.at[pl.ds(base + j * CHUNK, CHUNK)],
                    ssem,
                )
            )
        for s in stores:
            s.wait()

    return emb_kernel, nw, n_chunks


def kernel(x, table):
    emb_kernel, nw, n_chunks = _build()
    idx = x.astype(jnp.int32).reshape(nw, n_chunks, CHUNK)
    return emb_kernel(idx, table)
